# Initial kernel scaffold; baseline (speedup 1.0000x reference)
import jax
import jax.numpy as jnp
from jax.experimental import pallas as pl


def kernel(x, edge_index, edge_attr, W):
    raise NotImplementedError("write your pallas kernel here")


# speedup vs baseline: 9.2850x; 9.2850x over previous
import functools

import jax
import jax.numpy as jnp
from jax import lax
from jax.experimental import pallas as pl
from jax.experimental.pallas import tpu as pltpu
from jax.experimental.pallas import tpu_sc as plsc

N = 10000
E = 320000
D = 128
KT = 2
NLAYERS = 2

NC = 2
NS = 16
NW = NC * NS
CH = 128
NCHUNK = 79
EPADN = NW * NCHUNK * CH - E
NPAD = 10240
RPT = NPAD // NS

BN = 1000
NB = N // BN

_sc_mesh = plsc.VectorSubcoreMesh(
    core_axis_name="c", subcore_axis_name="s", num_cores=NC, num_subcores=NS
)


def _mm_body(x_ref, w_ref, o_ref):
    o_ref[...] = jnp.dot(x_ref[...], w_ref[0], preferred_element_type=jnp.float32)


def _project(x, Wt):
    return pl.pallas_call(
        _mm_body,
        grid=(KT, NB),
        in_specs=[
            pl.BlockSpec((BN, D), lambda k, i: (i, 0)),
            pl.BlockSpec((1, D, D), lambda k, i: (k, 0, 0)),
        ],
        out_specs=pl.BlockSpec((BN, D), lambda k, i: (k * NB + i, 0)),
        out_shape=jax.ShapeDtypeStruct((KT * N, D), jnp.float32),
    )(x, Wt)


@functools.partial(
    pl.kernel,
    out_type=jax.ShapeDtypeStruct((NC, NPAD, D), jnp.float32),
    mesh=_sc_mesh,
    scratch_types=[
        pltpu.VMEM((2, CH), jnp.int32),
        pltpu.VMEM((2, CH), jnp.int32),
        pltpu.VMEM((CH, D), jnp.float32),
        pltpu.VMEM((CH, D), jnp.float32),
        pltpu.VMEM_SHARED((NPAD, D), jnp.float32),
        pltpu.SemaphoreType.DMA,
        pltpu.SemaphoreType.DMA,
        pltpu.SemaphoreType.DMA,
        pltpu.SemaphoreType.DMA,
    ],
)
def _sc_edge(h_hbm, eidx_hbm, zer_hbm, out_hbm,
             ib0, ib1, rows0, rows1, acc_sh, isem0, isem1, gsem0, gsem1):
    cid = lax.axis_index("c")
    sid = lax.axis_index("s")
    wid = cid * NS + sid
    my_eidx = eidx_hbm.at[wid]

    pltpu.sync_copy(zer_hbm, acc_sh.at[pl.ds(sid * RPT, RPT)])
    plsc.subcore_barrier()

    pltpu.sync_copy(my_eidx.at[0], ib0)
    pltpu.async_copy(h_hbm.at[ib0.at[0]], rows0, gsem0)
    pltpu.async_copy(my_eidx.at[1], ib1, isem1)

    def step(c, ibA, ibB, rowsA, rowsB, isemA, isemB, gsemA, gsemB):
        @pl.when(c + 1 < NCHUNK)
        def _fire_next_gather():
            pltpu.make_async_copy(my_eidx.at[c + 1], ibB, isemB).wait()
            pltpu.async_copy(h_hbm.at[ibB.at[0]], rowsB, gsemB)

        pltpu.make_async_copy(h_hbm.at[ibA.at[0]], rowsA, gsemA).wait()
        pltpu.sync_copy(rowsA, acc_sh.at[ibA.at[1]], add=True)

        @pl.when(c + 2 < NCHUNK)
        def _prefetch_idx():
            pltpu.async_copy(my_eidx.at[c + 2], ibA, isemA)

    def body(c, carry):
        @pl.when(c % 2 == 0)
        def _even():
            step(c, ib0, ib1, rows0, rows1, isem0, isem1, gsem0, gsem1)

        @pl.when(c % 2 == 1)
        def _odd():
            step(c, ib1, ib0, rows1, rows0, isem1, isem0, gsem1, gsem0)

        return carry

    lax.fori_loop(0, NCHUNK, body, 0)
    plsc.subcore_barrier()

    pltpu.sync_copy(acc_sh.at[pl.ds(sid * RPT, RPT)],
                    out_hbm.at[cid].at[pl.ds(sid * RPT, RPT)])


def _comb_body(x_ref, p_ref, o_ref):
    s = p_ref[0] + p_ref[1]
    y = x_ref[...] + jnp.maximum(s, 0.0)
    nrm = jnp.sqrt(jnp.sum(y * y, axis=1, keepdims=True))
    o_ref[...] = y / jnp.maximum(nrm, 1e-12)


def _combine(x, part):
    return pl.pallas_call(
        _comb_body,
        grid=(NB,),
        in_specs=[
            pl.BlockSpec((BN, D), lambda i: (i, 0)),
            pl.BlockSpec((NC, BN, D), lambda i: (0, i, 0)),
        ],
        out_specs=pl.BlockSpec((BN, D), lambda i: (i, 0)),
        out_shape=jax.ShapeDtypeStruct((N, D), jnp.float32),
    )(x, part)


def kernel(x, edge_index, edge_attr, W):
    src = edge_index[0]
    dst = edge_index[1]
    gidx = (edge_attr - 1) * N + src
    gidxp = jnp.concatenate([gidx, jnp.zeros((EPADN,), jnp.int32)])
    dstp = jnp.concatenate([dst, jnp.full((EPADN,), N, jnp.int32)])
    eidx = jnp.stack(
        [gidxp.reshape(NW, NCHUNK, CH), dstp.reshape(NW, NCHUNK, CH)], axis=2
    )
    zer = jnp.zeros((RPT, D), jnp.float32)
    for t in range(NLAYERS):
        h = _project(x, W[t])
        part = _sc_edge(h, eidx, zer)
        x = _combine(x, part)
    return x


# speedup vs baseline: 18.5328x; 1.9960x over previous
import functools

import jax
import jax.numpy as jnp
from jax import lax
from jax.experimental import pallas as pl
from jax.experimental.pallas import tpu as pltpu
from jax.experimental.pallas import tpu_sc as plsc

N = 10000
E = 320000
D = 128
KT = 2
NLAYERS = 2

NC = 2
NS = 16
NW = NC * NS
CH = 128
NCHUNK = 79
EPADN = NW * NCHUNK * CH - E
NPAD = 10240
RPT = NPAD // NS

BN = 1000
NB = N // BN

_sc_mesh = plsc.VectorSubcoreMesh(
    core_axis_name="c", subcore_axis_name="s", num_cores=NC, num_subcores=NS
)


def _mm_body(x_ref, w_ref, o_ref):
    o_ref[...] = jnp.dot(x_ref[...], w_ref[0], preferred_element_type=jnp.float32)


def _project(x, Wt):
    return pl.pallas_call(
        _mm_body,
        grid=(KT, NB),
        in_specs=[
            pl.BlockSpec((BN, D), lambda k, i: (i, 0)),
            pl.BlockSpec((1, D, D), lambda k, i: (k, 0, 0)),
        ],
        out_specs=pl.BlockSpec((BN, D), lambda k, i: (k * NB + i, 0)),
        out_shape=jax.ShapeDtypeStruct((KT * N, D), jnp.float32),
    )(x, Wt)


@functools.partial(
    pl.kernel,
    out_type=jax.ShapeDtypeStruct((NC, NPAD, D), jnp.float32),
    mesh=_sc_mesh,
    scratch_types=[
        pltpu.VMEM((2, CH), jnp.int32),
        pltpu.VMEM((2, CH), jnp.int32),
        pltpu.VMEM((CH, D), jnp.float32),
        pltpu.VMEM((CH, D), jnp.float32),
        pltpu.VMEM_SHARED((NPAD, D), jnp.float32),
        pltpu.SemaphoreType.DMA,
        pltpu.SemaphoreType.DMA,
        pltpu.SemaphoreType.DMA,
        pltpu.SemaphoreType.DMA,
    ],
)
def _sc_edge(h_hbm, eidx_hbm, zer_hbm, out_hbm,
             ib0, ib1, rows0, rows1, acc_sh, isem0, isem1, gsem0, gsem1):
    cid = lax.axis_index("c")
    sid = lax.axis_index("s")
    wid = cid * NS + sid
    my_eidx = eidx_hbm.at[wid]

    pltpu.sync_copy(zer_hbm, acc_sh.at[pl.ds(sid * RPT, RPT)])
    plsc.subcore_barrier()

    pltpu.sync_copy(my_eidx.at[0], ib0)
    pltpu.async_copy(h_hbm.at[ib0.at[0]], rows0, gsem0)
    pltpu.async_copy(my_eidx.at[1], ib1, isem1)

    def step(c, ibA, ibB, rowsA, rowsB, isemA, isemB, gsemA, gsemB):
        @pl.when(c + 1 < NCHUNK)
        def _fire_next_gather():
            pltpu.make_async_copy(my_eidx.at[c + 1], ibB, isemB).wait()
            pltpu.async_copy(h_hbm.at[ibB.at[0]], rowsB, gsemB)

        pltpu.make_async_copy(h_hbm.at[ibA.at[0]], rowsA, gsemA).wait()
        pltpu.sync_copy(rowsA, acc_sh.at[ibA.at[1]], add=True)

        @pl.when(c + 2 < NCHUNK)
        def _prefetch_idx():
            pltpu.async_copy(my_eidx.at[c + 2], ibA, isemA)

    def body(c, carry):
        @pl.when(c % 2 == 0)
        def _even():
            step(c, ib0, ib1, rows0, rows1, isem0, isem1, gsem0, gsem1)

        @pl.when(c % 2 == 1)
        def _odd():
            step(c, ib1, ib0, rows1, rows0, isem1, isem0, gsem1, gsem0)

        return carry

    lax.fori_loop(0, NCHUNK, body, 0)
    plsc.subcore_barrier()

    pltpu.sync_copy(acc_sh.at[pl.ds(sid * RPT, RPT)],
                    out_hbm.at[cid].at[pl.ds(sid * RPT, RPT)])


def _comb_body(x_ref, p_ref, o_ref):
    s = p_ref[0] + p_ref[1]
    y = x_ref[...] + jnp.maximum(s, 0.0)
    nrm = jnp.sqrt(jnp.sum(y * y, axis=1, keepdims=True))
    o_ref[...] = y / jnp.maximum(nrm, 1e-12)


def _combine(x, part):
    return pl.pallas_call(
        _comb_body,
        grid=(NB,),
        in_specs=[
            pl.BlockSpec((BN, D), lambda i: (i, 0)),
            pl.BlockSpec((NC, BN, D), lambda i: (0, i, 0)),
        ],
        out_specs=pl.BlockSpec((BN, D), lambda i: (i, 0)),
        out_shape=jax.ShapeDtypeStruct((N, D), jnp.float32),
    )(x, part)


def kernel(x, edge_index, edge_attr, W):
    src = edge_index[0]
    dst = edge_index[1]
    gidx = (edge_attr - 1) * N + src
    ppt = EPADN // NW
    pad_g = jnp.broadcast_to(
        (jnp.arange(ppt, dtype=jnp.int32) * 128) % (KT * N), (NW, ppt)
    )
    pad_d = jnp.broadcast_to(
        N + (jnp.arange(ppt, dtype=jnp.int32) % (NPAD - N)), (NW, ppt)
    )
    gidxp = jnp.concatenate([gidx.reshape(NW, E // NW), pad_g], axis=1)
    dstp = jnp.concatenate([dst.reshape(NW, E // NW), pad_d], axis=1)
    eidx = jnp.stack(
        [gidxp.reshape(NW, NCHUNK, CH), dstp.reshape(NW, NCHUNK, CH)], axis=2
    )
    zer = jnp.zeros((RPT, D), jnp.float32)
    for t in range(NLAYERS):
        h = _project(x, W[t])
        part = _sc_edge(h, eidx, zer)
        x = _combine(x, part)
    return x
